# baseline (speedup 1.0000x reference)
import functools

import jax
import jax.numpy as jnp
from jax import lax
from jax.experimental import pallas as pl
from jax.experimental.pallas import tpu as pltpu
from jax.experimental.pallas import tpu_sc as plsc

D = 128
N = 10000
E = 320000

_info = plsc.get_sparse_core_info()
NC, NS = _info.num_cores, _info.num_subcores
NW = NC * NS
CB = 80
NP = 10240
NPT = NP // NS
NSTG = 64
ZR = 32

EWA, KGA, KLA = 4960, 2, 31
EWB, KGB, KLB = 5040, 3, 21
EA = NW * EWA
BE = 2560
NBA = EA // BE

_sc_mesh = plsc.VectorSubcoreMesh(core_axis_name="c", subcore_axis_name="s")


def _make_sc_gather(ew, kg, kl):
    eh = NW * ew

    @functools.partial(
        pl.kernel,
        out_type=jax.ShapeDtypeStruct((eh, D), jnp.float32),
        mesh=_sc_mesh,
        scratch_types=[
            pltpu.VMEM((kl, CB), jnp.int32),
            pltpu.VMEM((CB, D), jnp.float32),
            pltpu.VMEM((CB, D), jnp.float32),
            pltpu.VMEM((CB, D), jnp.float32),
            pltpu.VMEM((CB, D), jnp.float32),
            pltpu.SemaphoreType.DMA,
            pltpu.SemaphoreType.DMA,
            pltpu.SemaphoreType.DMA,
            pltpu.SemaphoreType.DMA,
            pltpu.SemaphoreType.DMA,
            pltpu.SemaphoreType.DMA,
            pltpu.SemaphoreType.DMA,
            pltpu.SemaphoreType.DMA,
        ],
    )
    def gather_k(xw_hbm, row4_hbm, g_hbm, idx_v, rows0_v, rows1_v, rows2_v,
                 rows3_v, gs0, gs1, gs2, gs3, ws0, ws1, ws2, ws3):
        c = lax.axis_index("c")
        s = lax.axis_index("s")
        wid = s * NC + c
        bufs = (rows0_v, rows1_v, rows2_v, rows3_v)
        gsems = (gs0, gs1, gs2, gs3)
        wsems = (ws0, ws1, ws2, ws3)

        def outer(jj, carry):
            pltpu.sync_copy(row4_hbm.at[wid].at[jj], idx_v)
            nq = kl // 4

            def quad(m, carry2):
                j0 = jj * kl + 4 * m
                gd = []
                for b in range(4):
                    gd.append(pltpu.async_copy(
                        xw_hbm.at[idx_v.at[4 * m + b]], bufs[b], gsems[b]))
                wd = []
                for b in range(4):
                    gd[b].wait()
                    wd.append(pltpu.async_copy(
                        bufs[b],
                        g_hbm.at[pl.ds(wid * ew + (j0 + b) * CB, CB)],
                        wsems[b]))
                for b in range(4):
                    wd[b].wait()
                return carry2

            lax.fori_loop(0, nq, quad, 0)
            for t in range(nq * 4, kl):
                jt = jj * kl + t
                pltpu.async_copy(xw_hbm.at[idx_v.at[t]], rows0_v, gs0).wait()
                pltpu.sync_copy(rows0_v,
                                g_hbm.at[pl.ds(wid * ew + jt * CB, CB)])
            return carry

        lax.fori_loop(0, kg, outer, 0)

    return gather_k


def _make_sc_scatter(ew, kg, kl):
    @functools.partial(
        pl.kernel,
        out_type=(
            jax.ShapeDtypeStruct((NC, NP, D), jnp.float32),
            jax.ShapeDtypeStruct((NW, NP), jnp.float32),
        ),
        mesh=_sc_mesh,
        compiler_params=pltpu.CompilerParams(needs_layout_passes=False),
        scratch_types=[
            pltpu.VMEM_SHARED((NP, D), jnp.float32),
            pltpu.VMEM((kl, CB), jnp.int32),
            pltpu.VMEM((CB, D), jnp.float32),
            pltpu.VMEM((CB, D), jnp.float32),
            pltpu.VMEM((NP,), jnp.float32),
            pltpu.VMEM((ZR, D), jnp.float32),
            pltpu.SemaphoreType.DMA,
            pltpu.SemaphoreType.DMA,
            pltpu.SemaphoreType.DMA,
            pltpu.SemaphoreType.DMA,
        ],
    )
    def scatter_k(oute_hbm, col4_hbm, z_hbm, part_hbm, cnt_hbm,
                  acc_sh, idx_v, upd0_v, upd1_v, cnt_v, zb_v,
                  ls0, ls1, as0, as1):
        c = lax.axis_index("c")
        s = lax.axis_index("s")
        wid = s * NC + c

        def init(i, carry):
            base = s * NPT + i * ZR
            pltpu.sync_copy(z_hbm.at[pl.ds(base, ZR)], zb_v)
            pltpu.sync_copy(zb_v, acc_sh.at[pl.ds(base, ZR)])
            return carry

        lax.fori_loop(0, NPT // ZR, init, 0)

        zvec = jnp.zeros((16,), jnp.float32)

        def zinit(i, carry):
            cnt_v[pl.ds(i * 16, 16)] = zvec
            return carry

        lax.fori_loop(0, NP // 16, zinit, 0)
        plsc.subcore_barrier()

        ones = jnp.full((16,), 1.0, jnp.float32)

        def counts(k):
            for v in range(CB // 16):
                cv = idx_v[k, pl.ds(v * 16, 16)]
                plsc.addupdate_scatter(cnt_v, [cv], ones)

        def outer(jj, carry):
            pltpu.sync_copy(col4_hbm.at[wid].at[jj], idx_v)

            def pair(m, carry2):
                j0 = jj * kl + 2 * m
                d0 = pltpu.async_copy(
                    oute_hbm.at[pl.ds(wid * ew + j0 * CB, CB)], upd0_v, ls0)
                d1 = pltpu.async_copy(
                    oute_hbm.at[pl.ds(wid * ew + (j0 + 1) * CB, CB)], upd1_v,
                    ls1)
                d0.wait()
                a0 = pltpu.async_copy(upd0_v, acc_sh.at[idx_v.at[2 * m]],
                                      as0, add=True)
                d1.wait()
                a1 = pltpu.async_copy(upd1_v, acc_sh.at[idx_v.at[2 * m + 1]],
                                      as1, add=True)
                counts(2 * m)
                counts(2 * m + 1)
                a0.wait()
                a1.wait()
                return carry2

            lax.fori_loop(0, kl // 2, pair, 0)
            jt = jj * kl + kl - 1
            pltpu.sync_copy(oute_hbm.at[pl.ds(wid * ew + jt * CB, CB)],
                            upd0_v)
            pltpu.sync_copy(upd0_v, acc_sh.at[idx_v.at[kl - 1]], add=True)
            counts(kl - 1)
            return carry

        lax.fori_loop(0, kg, outer, 0)
        plsc.subcore_barrier()

        def wout(i, carry):
            base = s * NPT + i * ZR
            pltpu.sync_copy(acc_sh.at[pl.ds(base, ZR)], zb_v)
            pltpu.sync_copy(zb_v, part_hbm.at[c].at[pl.ds(base, ZR)])
            return carry

        lax.fori_loop(0, NPT // ZR, wout, 0)
        pltpu.sync_copy(cnt_v, cnt_hbm.at[wid])

    return scatter_k


_sc_gather_a = _make_sc_gather(EWA, KGA, KLA)
_sc_gather_b = _make_sc_gather(EWB, KGB, KLB)
_sc_scatter_a = _make_sc_scatter(EWA, KGA, KLA)
_sc_scatter_b = _make_sc_scatter(EWB, KGB, KLB)


def _xw_body(x_ref, w_ref, o_ref):
    o_ref[...] = jnp.dot(x_ref[...], w_ref[...],
                         preferred_element_type=jnp.float32)


def _tc_xw(x, w):
    bn = 1024
    return pl.pallas_call(
        _xw_body,
        grid=(NP // bn,),
        in_specs=[
            pl.BlockSpec((bn, D), lambda i: (i, 0)),
            pl.BlockSpec((D, D), lambda i: (0, 0)),
        ],
        out_specs=pl.BlockSpec((bn, D), lambda i: (i, 0)),
        out_shape=jax.ShapeDtypeStruct((NP, D), jnp.float32),
    )(x, w)


def _mlp_body(g_ref, ea_ref, wa_ref, wb_ref, ba_ref, bb_ref, o_ref):
    pre = g_ref[...] + jnp.dot(ea_ref[...], wa_ref[...],
                               preferred_element_type=jnp.float32) + ba_ref[...]
    h = jnp.maximum(pre, 0.0)
    o_ref[...] = jnp.dot(h, wb_ref[...],
                         preferred_element_type=jnp.float32) + bb_ref[...]


def _tc_edge_mlp(g, ea, wa, wb, ba, bb, nblk, off):
    return pl.pallas_call(
        _mlp_body,
        grid=(nblk,),
        in_specs=[
            pl.BlockSpec((BE, D), lambda i: (i, 0)),
            pl.BlockSpec((BE, D), lambda i: (i + off, 0)),
            pl.BlockSpec((D, D), lambda i: (0, 0)),
            pl.BlockSpec((D, D), lambda i: (0, 0)),
            pl.BlockSpec((1, D), lambda i: (0, 0)),
            pl.BlockSpec((1, D), lambda i: (0, 0)),
        ],
        out_specs=pl.BlockSpec((BE, D), lambda i: (i, 0)),
        out_shape=jax.ShapeDtypeStruct((nblk * BE, D), jnp.float32),
    )(g, ea, wa, wb, ba, bb)


def _fin_body(x_ref, pa_ref, pb_ref, ca_ref, cb_ref, wa_ref, wm_ref, ba_ref,
              wb_ref, bb_ref, o_ref):
    ones_w = jnp.ones((NW, 1), jnp.float32)
    dn = (((0,), (0,)), ((), ()))
    cnt = (lax.dot_general(ca_ref[...], ones_w, dn,
                           preferred_element_type=jnp.float32)
           + lax.dot_general(cb_ref[...], ones_w, dn,
                             preferred_element_type=jnp.float32))
    denom = jnp.maximum(cnt, 1.0)
    sums = pa_ref[0] + pa_ref[1] + pb_ref[0] + pb_ref[1]
    mean = sums / denom
    pre = (jnp.dot(x_ref[...], wa_ref[...], preferred_element_type=jnp.float32)
           + jnp.dot(mean, wm_ref[...], preferred_element_type=jnp.float32)
           + ba_ref[...])
    h = jnp.maximum(pre, 0.0)
    o_ref[...] = jnp.dot(h, wb_ref[...],
                         preferred_element_type=jnp.float32) + bb_ref[...]


def _tc_final(x, pa, pb, ca, cb, wa, wm, ba, wb, bb):
    bn = 1024
    return pl.pallas_call(
        _fin_body,
        grid=(NP // bn,),
        in_specs=[
            pl.BlockSpec((bn, D), lambda i: (i, 0)),
            pl.BlockSpec((NC, bn, D), lambda i: (0, i, 0)),
            pl.BlockSpec((NC, bn, D), lambda i: (0, i, 0)),
            pl.BlockSpec((NW, bn), lambda i: (0, i)),
            pl.BlockSpec((NW, bn), lambda i: (0, i)),
            pl.BlockSpec((D, D), lambda i: (0, 0)),
            pl.BlockSpec((D, D), lambda i: (0, 0)),
            pl.BlockSpec((1, D), lambda i: (0, 0)),
            pl.BlockSpec((D, D), lambda i: (0, 0)),
            pl.BlockSpec((1, D), lambda i: (0, 0)),
        ],
        out_specs=pl.BlockSpec((bn, D), lambda i: (i, 0)),
        out_shape=jax.ShapeDtypeStruct((NP, D), jnp.float32),
    )(x, pa, pb, ca, cb, wa, wm, ba, wb, bb)


def kernel(x, edge_index, edge_attr, W1a, b1a, W1b, b1b, W2a, b2a, W2b, b2b):
    row = edge_index[0].astype(jnp.int32)
    col = edge_index[1].astype(jnp.int32)
    row4a = row[:EA].reshape(NW, KGA, KLA, CB)
    row4b = row[EA:].reshape(NW, KGB, KLB, CB)
    col4a = col[:EA].reshape(NW, KGA, KLA, CB)
    col4b = col[EA:].reshape(NW, KGB, KLB, CB)
    zeros = jnp.zeros((NP, D), jnp.float32)
    b1a2 = b1a.reshape(1, D)
    b1b2 = b1b.reshape(1, D)

    x_pad = jnp.pad(x, ((0, NP - N), (0, 0)))
    xw = _tc_xw(x_pad, W1a[:D])
    ga = _sc_gather_a(xw, row4a)
    gb = _sc_gather_b(xw, row4b)
    oea = _tc_edge_mlp(ga, edge_attr, W1a[D:], W1b, b1a2, b1b2, NBA, 0)
    oeb = _tc_edge_mlp(gb, edge_attr, W1a[D:], W1b, b1a2, b1b2,
                       E // BE - NBA, NBA)
    parta, cnta = _sc_scatter_a(oea, col4a, zeros)
    partb, cntb = _sc_scatter_b(oeb, col4b, zeros)
    out = _tc_final(x_pad, parta, partb, cnta, cntb, W2a[:D], W2a[D:],
                    b2a.reshape(1, D), W2b, b2b.reshape(1, D))
    return out[:N]


# speedup vs baseline: 1.0690x; 1.0690x over previous
import functools

import jax
import jax.numpy as jnp
from jax import lax
from jax.experimental import pallas as pl
from jax.experimental.pallas import tpu as pltpu
from jax.experimental.pallas import tpu_sc as plsc

D = 128
N = 10000
E = 320000

_info = plsc.get_sparse_core_info()
NC, NS = _info.num_cores, _info.num_subcores
NW = NC * NS
CB = 80
NP = 10240
NPT = NP // NS
NSTG = 64
ZR = 32

EWA, KGA, KLA = 4960, 2, 31
EWB, KGB, KLB = 5040, 3, 21
EA = NW * EWA
BE = 2560
NBA = EA // BE

_sc_mesh = plsc.VectorSubcoreMesh(core_axis_name="c", subcore_axis_name="s")


def _make_sc_gather(ew, kg, kl):
    eh = NW * ew

    @functools.partial(
        pl.kernel,
        out_type=jax.ShapeDtypeStruct((eh, D), jnp.float32),
        mesh=_sc_mesh,
        scratch_types=[
            pltpu.VMEM_SHARED((NP, D), jnp.float32),
            pltpu.VMEM((kl, CB), jnp.int32),
            pltpu.VMEM((CB, D), jnp.float32),
            pltpu.VMEM((CB, D), jnp.float32),
            pltpu.VMEM((CB, D), jnp.float32),
            pltpu.VMEM((NSTG, D), jnp.float32),
            pltpu.SemaphoreType.DMA,
            pltpu.SemaphoreType.DMA,
            pltpu.SemaphoreType.DMA,
            pltpu.SemaphoreType.DMA,
            pltpu.SemaphoreType.DMA,
            pltpu.SemaphoreType.DMA,
        ],
    )
    def gather_k(xw_hbm, row4_hbm, g_hbm, table_sh, idx_v, rows0_v, rows1_v,
                 rows2_v, stg_v, gs0, gs1, gs2, ws0, ws1, ws2):
        c = lax.axis_index("c")
        s = lax.axis_index("s")
        wid = s * NC + c
        bufs = (rows0_v, rows1_v, rows2_v)
        gsems = (gs0, gs1, gs2)
        wsems = (ws0, ws1, ws2)

        def stage(i, carry):
            base = s * NPT + i * NSTG
            pltpu.sync_copy(xw_hbm.at[pl.ds(base, NSTG)], stg_v)
            pltpu.sync_copy(stg_v, table_sh.at[pl.ds(base, NSTG)])
            return carry

        lax.fori_loop(0, NPT // NSTG, stage, 0)
        plsc.subcore_barrier()

        def outer(jj, carry):
            pltpu.sync_copy(row4_hbm.at[wid].at[jj], idx_v)
            nt = kl // 3

            def triple(m, carry2):
                j0 = jj * kl + 3 * m
                gd = [pltpu.async_copy(table_sh.at[idx_v.at[3 * m + b]],
                                       bufs[b], gsems[b]) for b in range(3)]
                wd = []
                for b in range(3):
                    gd[b].wait()
                    wd.append(pltpu.async_copy(
                        bufs[b],
                        g_hbm.at[pl.ds(wid * ew + (j0 + b) * CB, CB)],
                        wsems[b]))
                for b in range(3):
                    wd[b].wait()
                return carry2

            lax.fori_loop(0, nt, triple, 0)
            for t in range(nt * 3, kl):
                jt = jj * kl + t
                pltpu.async_copy(table_sh.at[idx_v.at[t]], rows0_v,
                                 gs0).wait()
                pltpu.sync_copy(rows0_v,
                                g_hbm.at[pl.ds(wid * ew + jt * CB, CB)])
            return carry

        lax.fori_loop(0, kg, outer, 0)

    return gather_k


def _make_sc_scatter(ew, kg, kl):
    @functools.partial(
        pl.kernel,
        out_type=(
            jax.ShapeDtypeStruct((NC, NP, D), jnp.float32),
            jax.ShapeDtypeStruct((NW, NP), jnp.float32),
        ),
        mesh=_sc_mesh,
        compiler_params=pltpu.CompilerParams(needs_layout_passes=False),
        scratch_types=[
            pltpu.VMEM_SHARED((NP, D), jnp.float32),
            pltpu.VMEM((kl, CB), jnp.int32),
            pltpu.VMEM((CB, D), jnp.float32),
            pltpu.VMEM((CB, D), jnp.float32),
            pltpu.VMEM((NP,), jnp.float32),
            pltpu.VMEM((ZR, D), jnp.float32),
            pltpu.SemaphoreType.DMA,
            pltpu.SemaphoreType.DMA,
            pltpu.SemaphoreType.DMA,
            pltpu.SemaphoreType.DMA,
        ],
    )
    def scatter_k(oute_hbm, col4_hbm, z_hbm, part_hbm, cnt_hbm,
                  acc_sh, idx_v, upd0_v, upd1_v, cnt_v, zb_v,
                  ls0, ls1, as0, as1):
        c = lax.axis_index("c")
        s = lax.axis_index("s")
        wid = s * NC + c

        def init(i, carry):
            base = s * NPT + i * ZR
            pltpu.sync_copy(z_hbm.at[pl.ds(base, ZR)], zb_v)
            pltpu.sync_copy(zb_v, acc_sh.at[pl.ds(base, ZR)])
            return carry

        lax.fori_loop(0, NPT // ZR, init, 0)

        zvec = jnp.zeros((16,), jnp.float32)

        def zinit(i, carry):
            cnt_v[pl.ds(i * 16, 16)] = zvec
            return carry

        lax.fori_loop(0, NP // 16, zinit, 0)
        plsc.subcore_barrier()

        ones = jnp.full((16,), 1.0, jnp.float32)

        def counts(k):
            for v in range(CB // 16):
                cv = idx_v[k, pl.ds(v * 16, 16)]
                plsc.addupdate_scatter(cnt_v, [cv], ones)

        def outer(jj, carry):
            pltpu.sync_copy(col4_hbm.at[wid].at[jj], idx_v)

            def pair(m, carry2):
                j0 = jj * kl + 2 * m
                d0 = pltpu.async_copy(
                    oute_hbm.at[pl.ds(wid * ew + j0 * CB, CB)], upd0_v, ls0)
                d1 = pltpu.async_copy(
                    oute_hbm.at[pl.ds(wid * ew + (j0 + 1) * CB, CB)], upd1_v,
                    ls1)
                d0.wait()
                a0 = pltpu.async_copy(upd0_v, acc_sh.at[idx_v.at[2 * m]],
                                      as0, add=True)
                d1.wait()
                a1 = pltpu.async_copy(upd1_v, acc_sh.at[idx_v.at[2 * m + 1]],
                                      as1, add=True)
                counts(2 * m)
                counts(2 * m + 1)
                a0.wait()
                a1.wait()
                return carry2

            lax.fori_loop(0, kl // 2, pair, 0)
            jt = jj * kl + kl - 1
            pltpu.sync_copy(oute_hbm.at[pl.ds(wid * ew + jt * CB, CB)],
                            upd0_v)
            pltpu.sync_copy(upd0_v, acc_sh.at[idx_v.at[kl - 1]], add=True)
            counts(kl - 1)
            return carry

        lax.fori_loop(0, kg, outer, 0)
        plsc.subcore_barrier()

        def wout(i, carry):
            base = s * NPT + i * ZR
            pltpu.sync_copy(acc_sh.at[pl.ds(base, ZR)], zb_v)
            pltpu.sync_copy(zb_v, part_hbm.at[c].at[pl.ds(base, ZR)])
            return carry

        lax.fori_loop(0, NPT // ZR, wout, 0)
        pltpu.sync_copy(cnt_v, cnt_hbm.at[wid])

    return scatter_k


_sc_gather_a = _make_sc_gather(EWA, KGA, KLA)
_sc_gather_b = _make_sc_gather(EWB, KGB, KLB)
_sc_scatter_a = _make_sc_scatter(EWA, KGA, KLA)
_sc_scatter_b = _make_sc_scatter(EWB, KGB, KLB)


def _xw_body(x_ref, w_ref, o_ref):
    o_ref[...] = jnp.dot(x_ref[...], w_ref[...],
                         preferred_element_type=jnp.float32)


def _tc_xw(x, w):
    bn = 1024
    return pl.pallas_call(
        _xw_body,
        grid=(NP // bn,),
        in_specs=[
            pl.BlockSpec((bn, D), lambda i: (i, 0)),
            pl.BlockSpec((D, D), lambda i: (0, 0)),
        ],
        out_specs=pl.BlockSpec((bn, D), lambda i: (i, 0)),
        out_shape=jax.ShapeDtypeStruct((NP, D), jnp.float32),
    )(x, w)


def _mlp_body(g_ref, ea_ref, wa_ref, wb_ref, ba_ref, bb_ref, o_ref):
    pre = g_ref[...] + jnp.dot(ea_ref[...], wa_ref[...],
                               preferred_element_type=jnp.float32) + ba_ref[...]
    h = jnp.maximum(pre, 0.0)
    o_ref[...] = jnp.dot(h, wb_ref[...],
                         preferred_element_type=jnp.float32) + bb_ref[...]


def _tc_edge_mlp(g, ea, wa, wb, ba, bb, nblk, off):
    return pl.pallas_call(
        _mlp_body,
        grid=(nblk,),
        in_specs=[
            pl.BlockSpec((BE, D), lambda i: (i, 0)),
            pl.BlockSpec((BE, D), lambda i: (i + off, 0)),
            pl.BlockSpec((D, D), lambda i: (0, 0)),
            pl.BlockSpec((D, D), lambda i: (0, 0)),
            pl.BlockSpec((1, D), lambda i: (0, 0)),
            pl.BlockSpec((1, D), lambda i: (0, 0)),
        ],
        out_specs=pl.BlockSpec((BE, D), lambda i: (i, 0)),
        out_shape=jax.ShapeDtypeStruct((nblk * BE, D), jnp.float32),
    )(g, ea, wa, wb, ba, bb)


def _fin_body(x_ref, pa_ref, pb_ref, ca_ref, cb_ref, wa_ref, wm_ref, ba_ref,
              wb_ref, bb_ref, o_ref):
    ones_w = jnp.ones((NW, 1), jnp.float32)
    dn = (((0,), (0,)), ((), ()))
    cnt = (lax.dot_general(ca_ref[...], ones_w, dn,
                           preferred_element_type=jnp.float32)
           + lax.dot_general(cb_ref[...], ones_w, dn,
                             preferred_element_type=jnp.float32))
    denom = jnp.maximum(cnt, 1.0)
    sums = pa_ref[0] + pa_ref[1] + pb_ref[0] + pb_ref[1]
    mean = sums / denom
    pre = (jnp.dot(x_ref[...], wa_ref[...], preferred_element_type=jnp.float32)
           + jnp.dot(mean, wm_ref[...], preferred_element_type=jnp.float32)
           + ba_ref[...])
    h = jnp.maximum(pre, 0.0)
    o_ref[...] = jnp.dot(h, wb_ref[...],
                         preferred_element_type=jnp.float32) + bb_ref[...]


def _tc_final(x, pa, pb, ca, cb, wa, wm, ba, wb, bb):
    bn = 1024
    return pl.pallas_call(
        _fin_body,
        grid=(NP // bn,),
        in_specs=[
            pl.BlockSpec((bn, D), lambda i: (i, 0)),
            pl.BlockSpec((NC, bn, D), lambda i: (0, i, 0)),
            pl.BlockSpec((NC, bn, D), lambda i: (0, i, 0)),
            pl.BlockSpec((NW, bn), lambda i: (0, i)),
            pl.BlockSpec((NW, bn), lambda i: (0, i)),
            pl.BlockSpec((D, D), lambda i: (0, 0)),
            pl.BlockSpec((D, D), lambda i: (0, 0)),
            pl.BlockSpec((1, D), lambda i: (0, 0)),
            pl.BlockSpec((D, D), lambda i: (0, 0)),
            pl.BlockSpec((1, D), lambda i: (0, 0)),
        ],
        out_specs=pl.BlockSpec((bn, D), lambda i: (i, 0)),
        out_shape=jax.ShapeDtypeStruct((NP, D), jnp.float32),
    )(x, pa, pb, ca, cb, wa, wm, ba, wb, bb)


def kernel(x, edge_index, edge_attr, W1a, b1a, W1b, b1b, W2a, b2a, W2b, b2b):
    row = edge_index[0].astype(jnp.int32)
    col = edge_index[1].astype(jnp.int32)
    row4a = row[:EA].reshape(NW, KGA, KLA, CB)
    row4b = row[EA:].reshape(NW, KGB, KLB, CB)
    col4a = col[:EA].reshape(NW, KGA, KLA, CB)
    col4b = col[EA:].reshape(NW, KGB, KLB, CB)
    zeros = jnp.zeros((NP, D), jnp.float32)
    b1a2 = b1a.reshape(1, D)
    b1b2 = b1b.reshape(1, D)

    x_pad = jnp.pad(x, ((0, NP - N), (0, 0)))
    xw = _tc_xw(x_pad, W1a[:D])
    ga = _sc_gather_a(xw, row4a)
    gb = _sc_gather_b(xw, row4b)
    oea = _tc_edge_mlp(ga, edge_attr, W1a[D:], W1b, b1a2, b1b2, NBA, 0)
    oeb = _tc_edge_mlp(gb, edge_attr, W1a[D:], W1b, b1a2, b1b2,
                       E // BE - NBA, NBA)
    parta, cnta = _sc_scatter_a(oea, col4a, zeros)
    partb, cntb = _sc_scatter_b(oeb, col4b, zeros)
    out = _tc_final(x_pad, parta, partb, cnta, cntb, W2a[:D], W2a[D:],
                    b2a.reshape(1, D), W2b, b2b.reshape(1, D))
    return out[:N]
